# baseline (speedup 1.0000x reference)
import functools

import jax
import jax.numpy as jnp
from jax import lax
from jax.experimental import pallas as pl
from jax.experimental.pallas import tpu as pltpu
from jax.experimental.pallas import tpu_sc as plsc

DIM = 256
KNN = 16
NLAYERS = 3

_F32 = jnp.float32


def _topk_body(pos_ref, posT_ref, out_ref, *, n):
    b = pl.program_id(0)
    prow = pos_ref[0]
    pcol = posT_ref[0]
    inner = (prow[:, 0:1] * pcol[0:1, :]
             + prow[:, 1:2] * pcol[1:2, :]
             + prow[:, 2:3] * pcol[2:3, :])
    xxr = prow[:, 0:1] ** 2 + prow[:, 1:2] ** 2 + prow[:, 2:3] ** 2
    xxc = pcol[0:1, :] ** 2 + pcol[1:2, :] ** 2 + pcol[2:3, :] ** 2
    pd = 2.0 * inner - xxr - xxc
    br = prow.shape[0]
    cols = lax.broadcasted_iota(jnp.int32, (br, n), 1)
    bits = lax.bitcast_convert_type(pd, jnp.int32)
    minint = jnp.int32(-(2 ** 31))
    skey = jnp.where(bits < 0, jnp.invert(bits) ^ minint, bits)
    ck = (skey & jnp.int32(-65536)) | (jnp.int32(n - 1) - cols)
    outs = []
    for _ in range(KNN):
        m = jnp.max(ck, axis=1, keepdims=True)
        idx = jnp.int32(n - 1) - (m & jnp.int32(0xFFFF))
        outs.append(idx)
        ck = jnp.where(ck == m, minint, ck)
    out_ref[0] = jnp.concatenate(outs, axis=1) + b * n


def _topk(pos8, posT, n, br=256):
    bsz = pos8.shape[0]
    return pl.pallas_call(
        functools.partial(_topk_body, n=n),
        grid=(bsz, n // br),
        in_specs=[
            pl.BlockSpec((1, br, 8), lambda b, i: (b, i, 0)),
            pl.BlockSpec((1, 8, n), lambda b, i: (b, 0, 0)),
        ],
        out_specs=pl.BlockSpec((1, br, KNN), lambda b, i: (b, i, 0)),
        out_shape=jax.ShapeDtypeStruct((bsz, n, KNN), jnp.int32),
    )(pos8, posT)


def _emb_body(pts_ref, w_ref, b_ref, out_ref):
    out_ref[...] = (jnp.dot(pts_ref[...], w_ref[...],
                            preferred_element_type=_F32) + b_ref[...])


def _emb(pts8, w8, b2, blk=512):
    rows = pts8.shape[0]
    return pl.pallas_call(
        _emb_body,
        grid=(rows // blk,),
        in_specs=[
            pl.BlockSpec((blk, 8), lambda i: (i, 0)),
            pl.BlockSpec((8, DIM), lambda i: (0, 0)),
            pl.BlockSpec((1, DIM), lambda i: (0, 0)),
        ],
        out_specs=pl.BlockSpec((blk, DIM), lambda i: (i, 0)),
        out_shape=jax.ShapeDtypeStruct((rows, DIM), _F32),
    )(pts8, w8, b2)


def _kv_body(x_ref, wk_ref, wv_ref, out_ref):
    x = x_ref[...].astype(jnp.bfloat16)
    k = jnp.dot(x, wk_ref[...], preferred_element_type=_F32)
    v = jnp.dot(x, wv_ref[...], preferred_element_type=_F32)
    out_ref[...] = jnp.concatenate([k, v], axis=1).astype(jnp.bfloat16)


def _kv(x, wk, wv, blk=512):
    rows = x.shape[0]
    return pl.pallas_call(
        _kv_body,
        grid=(rows // blk,),
        in_specs=[
            pl.BlockSpec((blk, DIM), lambda i: (i, 0)),
            pl.BlockSpec((DIM, DIM), lambda i: (0, 0)),
            pl.BlockSpec((DIM, DIM), lambda i: (0, 0)),
        ],
        out_specs=pl.BlockSpec((blk, 2 * DIM), lambda i: (i, 0)),
        out_shape=jax.ShapeDtypeStruct((rows, 2 * DIM), jnp.bfloat16),
    )(x, wk, wv)


def _sc_gather(table, idx):
    tot = idx.shape[0]
    row_shape = table.shape[1:]
    info = plsc.get_sparse_core_info()
    nw = info.num_cores * info.num_subcores
    rows_w = tot // nw
    ch = 128
    nc = rows_w // ch
    mesh = plsc.VectorSubcoreMesh(core_axis_name="c", subcore_axis_name="s")

    @functools.partial(
        pl.kernel, mesh=mesh,
        out_type=jax.ShapeDtypeStruct((tot,) + row_shape, table.dtype),
        scratch_types=[
            pltpu.VMEM((rows_w,), jnp.int32),
            pltpu.VMEM((ch,) + row_shape, table.dtype),
            pltpu.SemaphoreType.DMA,
        ],
    )
    def gk(idx_hbm, table_hbm, out_hbm, idx_v, buf, sem):
        wid = lax.axis_index("s") * info.num_cores + lax.axis_index("c")
        base = wid * rows_w
        pltpu.sync_copy(idx_hbm.at[pl.ds(base, rows_w)], idx_v)

        def body(c, carry):
            off = c * ch
            pltpu.async_copy(table_hbm.at[idx_v.at[pl.ds(off, ch)]],
                             buf, sem).wait()
            pltpu.sync_copy(buf, out_hbm.at[pl.ds(base + off, ch)])
            return carry

        lax.fori_loop(0, nc, body, 0)

    return gk(idx, table)


def _pair_body(kvf_ref, posg_ref, pos_ref, x_ref,
               wq_ref, pw1_ref, pb1_ref, pw2_ref, pb2_ref,
               aw1_ref, ab1_ref, aw2_ref, ab2_ref, ow_ref, ob_ref,
               out_ref, *, br):
    pr = br * KNN
    kf = kvf_ref[:, :DIM]
    vf = kvf_ref[:, DIM:]
    rel3 = (posg_ref[...].reshape(br, KNN, 128)
            - pos_ref[...].reshape(br, 1, 128))
    rel = rel3.reshape(pr, 128).astype(jnp.bfloat16)
    h = jnp.maximum(jnp.dot(rel, pw1_ref[...],
                            preferred_element_type=_F32) + pb1_ref[...], 0.0)
    pe = (jnp.dot(h.astype(jnp.bfloat16), pw2_ref[...],
                  preferred_element_type=_F32) + pb2_ref[...])
    x = x_ref[...]
    q = jnp.dot(x.astype(jnp.bfloat16), wq_ref[...],
                preferred_element_type=_F32)
    energy = (q.reshape(br, 1, DIM)
              - kf.astype(_F32).reshape(br, KNN, DIM)
              + pe.reshape(br, KNN, DIM)).reshape(pr, DIM)
    a = jnp.maximum(jnp.dot(energy.astype(jnp.bfloat16), aw1_ref[...],
                            preferred_element_type=_F32) + ab1_ref[...], 0.0)
    a = (jnp.dot(a.astype(jnp.bfloat16), aw2_ref[...],
                 preferred_element_type=_F32) + ab2_ref[...])
    a3 = a.reshape(br, KNN, DIM)
    m = jnp.max(a3, axis=1, keepdims=True)
    e = jnp.exp(a3 - m)
    w = e / jnp.sum(e, axis=1, keepdims=True)
    out = jnp.sum(w * vf.astype(_F32).reshape(br, KNN, DIM),
                  axis=1)
    res = jnp.maximum(jnp.dot(out.astype(jnp.bfloat16), ow_ref[...],
                              preferred_element_type=_F32) + ob_ref[...], 0.0)
    out_ref[...] = x + res


def _pair(kvf, posg, pos16, x, wq, pw1, pb1, pw2, pb2,
          aw1, ab1, aw2, ab2, ow, ob, br=128):
    rows = x.shape[0]
    wspec = pl.BlockSpec((DIM, DIM), lambda i: (0, 0))
    bspec = pl.BlockSpec((1, DIM), lambda i: (0, 0))
    return pl.pallas_call(
        functools.partial(_pair_body, br=br),
        grid=(rows // br,),
        in_specs=[
            pl.BlockSpec((br * KNN, 2 * DIM), lambda i: (i, 0)),
            pl.BlockSpec((br * KNN, 128), lambda i: (i, 0)),
            pl.BlockSpec((br, 128), lambda i: (i, 0)),
            pl.BlockSpec((br, DIM), lambda i: (i, 0)),
            wspec,
            pl.BlockSpec((128, DIM), lambda i: (0, 0)), bspec,
            wspec, bspec, wspec, bspec, wspec, bspec, wspec, bspec,
        ],
        out_specs=pl.BlockSpec((br, DIM), lambda i: (i, 0)),
        out_shape=jax.ShapeDtypeStruct((rows, DIM), _F32),
    )(kvf, posg, pos16, x, wq, pw1, pb1, pw2, pb2,
      aw1, ab1, aw2, ab2, ow, ob)


def _fin_body(x_ref, g_ref, b_ref, fw_ref, fb_ref, out_ref, *, n):
    x = x_ref[0]
    mu = jnp.mean(x, axis=1, keepdims=True)
    var = jnp.mean((x - mu) ** 2, axis=1, keepdims=True)
    xn = (x - mu) / jnp.sqrt(var + 1e-5) * g_ref[...] + b_ref[...]
    mean = jnp.sum(xn, axis=0, keepdims=True) * _F32(1.0 / n)
    z = jnp.sum(mean * fw_ref[...], axis=1, keepdims=True) + fb_ref[...]
    out_ref[...] = jnp.broadcast_to(1.0 / (1.0 + jnp.exp(-z)), (1, 1, 128))


def _fin(x3, g2, b2, fw2, fb2):
    bsz, n, _ = x3.shape
    return pl.pallas_call(
        functools.partial(_fin_body, n=n),
        grid=(bsz,),
        in_specs=[
            pl.BlockSpec((1, n, DIM), lambda b: (b, 0, 0)),
            pl.BlockSpec((1, DIM), lambda b: (0, 0)),
            pl.BlockSpec((1, DIM), lambda b: (0, 0)),
            pl.BlockSpec((1, DIM), lambda b: (0, 0)),
            pl.BlockSpec((1, 1), lambda b: (0, 0)),
        ],
        out_specs=pl.BlockSpec((1, 1, 128), lambda b: (b, 0, 0)),
        out_shape=jax.ShapeDtypeStruct((bsz, 1, 128), _F32),
    )(x3, g2, b2, fw2, fb2)


def kernel(vector_field, pathline_src, params):
    del vector_field
    bsz, ll, kk, c = pathline_src.shape
    n = ll * kk
    p = params
    pts = pathline_src.reshape(bsz, n, c)
    pos = pts[..., :3]
    pos8 = jnp.pad(pos, ((0, 0), (0, 0), (0, 5)))
    posT = pos8.transpose(0, 2, 1)
    pos128 = jnp.pad(pos, ((0, 0), (0, 0), (0, 125))).reshape(bsz * n, 128)
    pts8 = jnp.pad(pts, ((0, 0), (0, 0), (0, 8 - c))).reshape(bsz * n, 8)

    knn = _topk(pos8, posT, n)
    idx_flat = knn.reshape(bsz * n * KNN)

    posg = _sc_gather(pos128, idx_flat)

    w8 = jnp.pad(p['emb_w'], ((0, 8 - c), (0, 0)))
    x = _emb(pts8, w8, p['emb_b'].reshape(1, DIM))

    bf = jnp.bfloat16
    for i in range(NLAYERS):
        kv = _kv(x, p['wk'][i].astype(bf), p['wv'][i].astype(bf))
        kv32 = lax.bitcast_convert_type(
            kv.reshape(bsz * n, DIM, 2), jnp.int32)
        kvf32 = _sc_gather(kv32, idx_flat)
        kvf = lax.bitcast_convert_type(
            kvf32, jnp.bfloat16).reshape(bsz * n * KNN, 2 * DIM)
        pw1 = jnp.pad(p['pos_w1'][i], ((0, 125), (0, 0)))
        x = _pair(kvf, posg, pos128, x,
                  p['wq'][i].astype(bf), pw1.astype(bf),
                  p['pos_b1'][i].reshape(1, DIM),
                  p['pos_w2'][i].astype(bf), p['pos_b2'][i].reshape(1, DIM),
                  p['attn_w1'][i].astype(bf), p['attn_b1'][i].reshape(1, DIM),
                  p['attn_w2'][i].astype(bf), p['attn_b2'][i].reshape(1, DIM),
                  p['out_w'][i].astype(bf), p['out_b'][i].reshape(1, DIM))

    out = _fin(x.reshape(bsz, n, DIM), p['ln_g'].reshape(1, DIM),
               p['ln_b'].reshape(1, DIM), p['fc_w'].reshape(1, DIM),
               p['fc_b'].reshape(1, 1))
    return out[:, 0, :1]


# speedup vs baseline: 3.3960x; 3.3960x over previous
import functools

import jax
import jax.numpy as jnp
from jax import lax
from jax.experimental import pallas as pl
from jax.experimental.pallas import tpu as pltpu
from jax.experimental.pallas import tpu_sc as plsc

DIM = 256
KNN = 16
NLAYERS = 3

_F32 = jnp.float32


def _topk_body(pos_ref, posT_ref, out_ref, *, n):
    b = pl.program_id(0)
    prow = pos_ref[0]
    pcol = posT_ref[0]
    inner = (prow[:, 0:1] * pcol[0:1, :]
             + prow[:, 1:2] * pcol[1:2, :]
             + prow[:, 2:3] * pcol[2:3, :])
    xxr = prow[:, 0:1] ** 2 + prow[:, 1:2] ** 2 + prow[:, 2:3] ** 2
    xxc = pcol[0:1, :] ** 2 + pcol[1:2, :] ** 2 + pcol[2:3, :] ** 2
    pd = 2.0 * inner - xxr - xxc
    br = prow.shape[0]
    cols = lax.broadcasted_iota(jnp.int32, (br, n), 1)
    bits = lax.bitcast_convert_type(pd, jnp.int32)
    minint = jnp.int32(-(2 ** 31))
    skey = jnp.where(bits < 0, jnp.invert(bits) ^ minint, bits)
    ck = (skey & jnp.int32(-65536)) | (jnp.int32(n - 1) - cols)
    outs = []
    for _ in range(KNN):
        m = jnp.max(ck, axis=1, keepdims=True)
        idx = jnp.int32(n - 1) - (m & jnp.int32(0xFFFF))
        outs.append(idx)
        ck = jnp.where(ck == m, minint, ck)
    out_ref[0] = jnp.concatenate(outs, axis=1) + b * n


def _topk(pos8, posT, n, br=256):
    bsz = pos8.shape[0]
    return pl.pallas_call(
        functools.partial(_topk_body, n=n),
        grid=(bsz, n // br),
        in_specs=[
            pl.BlockSpec((1, br, 8), lambda b, i: (b, i, 0)),
            pl.BlockSpec((1, 8, n), lambda b, i: (b, 0, 0)),
        ],
        out_specs=pl.BlockSpec((1, br, KNN), lambda b, i: (b, i, 0)),
        out_shape=jax.ShapeDtypeStruct((bsz, n, KNN), jnp.int32),
    )(pos8, posT)


def _emb_body(pts_ref, w_ref, b_ref, out_ref):
    out_ref[...] = (jnp.dot(pts_ref[...], w_ref[...],
                            preferred_element_type=_F32) + b_ref[...])


def _emb(pts8, w8, b2, blk=512):
    rows = pts8.shape[0]
    return pl.pallas_call(
        _emb_body,
        grid=(rows // blk,),
        in_specs=[
            pl.BlockSpec((blk, 8), lambda i: (i, 0)),
            pl.BlockSpec((8, DIM), lambda i: (0, 0)),
            pl.BlockSpec((1, DIM), lambda i: (0, 0)),
        ],
        out_specs=pl.BlockSpec((blk, DIM), lambda i: (i, 0)),
        out_shape=jax.ShapeDtypeStruct((rows, DIM), _F32),
    )(pts8, w8, b2)


def _rtne16(f):
    u = lax.bitcast_convert_type(f, jnp.uint32)
    return (u + jnp.uint32(0x7FFF) + ((u >> 16) & jnp.uint32(1))) >> 16


def _kv_body(x_ref, wk_ref, wv_ref, out_ref):
    x = x_ref[...]
    k = jnp.dot(x, wk_ref[...], preferred_element_type=_F32)
    v = jnp.dot(x, wv_ref[...], preferred_element_type=_F32)
    packed = (_rtne16(v) << 16) | _rtne16(k)
    out_ref[...] = lax.bitcast_convert_type(packed, jnp.int32)


def _kv(x, wk, wv, blk=512):
    rows = x.shape[0]
    return pl.pallas_call(
        _kv_body,
        grid=(rows // blk,),
        in_specs=[
            pl.BlockSpec((blk, DIM), lambda i: (i, 0)),
            pl.BlockSpec((DIM, DIM), lambda i: (0, 0)),
            pl.BlockSpec((DIM, DIM), lambda i: (0, 0)),
        ],
        out_specs=pl.BlockSpec((blk, DIM), lambda i: (i, 0)),
        out_shape=jax.ShapeDtypeStruct((rows, DIM), jnp.int32),
    )(x, wk, wv)


def _sc_gather(table, idx):
    tot = idx.shape[0]
    row_shape = table.shape[1:]
    info = plsc.get_sparse_core_info()
    nw = info.num_cores * info.num_subcores
    rows_w = tot // nw
    ch = 128
    nc = rows_w // ch
    mesh = plsc.VectorSubcoreMesh(core_axis_name="c", subcore_axis_name="s")

    @functools.partial(
        pl.kernel, mesh=mesh,
        out_type=jax.ShapeDtypeStruct((tot,) + row_shape, table.dtype),
        scratch_types=[
            pltpu.VMEM((rows_w,), jnp.int32),
            pltpu.VMEM((ch,) + row_shape, table.dtype),
            pltpu.SemaphoreType.DMA,
        ],
    )
    def gk(idx_hbm, table_hbm, out_hbm, idx_v, buf, sem):
        wid = lax.axis_index("s") * info.num_cores + lax.axis_index("c")
        base = wid * rows_w
        pltpu.sync_copy(idx_hbm.at[pl.ds(base, rows_w)], idx_v)

        def body(c, carry):
            off = c * ch
            pltpu.async_copy(table_hbm.at[idx_v.at[pl.ds(off, ch)]],
                             buf, sem).wait()
            pltpu.sync_copy(buf, out_hbm.at[pl.ds(base + off, ch)])
            return carry

        lax.fori_loop(0, nc, body, 0)

    return gk(idx, table)


def _pair_body(kvf_ref, posg_ref, pos_ref, x_ref,
               wq_ref, pw1_ref, pb1_ref, pw2_ref, pb2_ref,
               aw1_ref, ab1_ref, aw2_ref, ab2_ref, ow_ref, ob_ref,
               out_ref, *, br):
    pr = br * KNN
    kw = kvf_ref[...]
    kf = lax.bitcast_convert_type(kw << 16, _F32)
    vf = lax.bitcast_convert_type(kw & jnp.int32(-65536), _F32)
    rel3 = (posg_ref[...].reshape(br, KNN, 128)
            - pos_ref[...].reshape(br, 1, 128))
    rel = rel3.reshape(pr, 128)
    h = jnp.maximum(jnp.dot(rel, pw1_ref[...],
                            preferred_element_type=_F32) + pb1_ref[...], 0.0)
    pe = jnp.dot(h, pw2_ref[...], preferred_element_type=_F32) + pb2_ref[...]
    x = x_ref[...]
    q = jnp.dot(x, wq_ref[...], preferred_element_type=_F32)
    energy = (q.reshape(br, 1, DIM)
              - kf.reshape(br, KNN, DIM)
              + pe.reshape(br, KNN, DIM)).reshape(pr, DIM)
    a = jnp.maximum(jnp.dot(energy, aw1_ref[...],
                            preferred_element_type=_F32) + ab1_ref[...], 0.0)
    a = jnp.dot(a, aw2_ref[...], preferred_element_type=_F32) + ab2_ref[...]
    a3 = a.reshape(br, KNN, DIM)
    m = jnp.max(a3, axis=1, keepdims=True)
    e = jnp.exp(a3 - m)
    w = e / jnp.sum(e, axis=1, keepdims=True)
    out = jnp.sum(w * vf.reshape(br, KNN, DIM), axis=1)
    res = jnp.maximum(jnp.dot(out, ow_ref[...],
                              preferred_element_type=_F32) + ob_ref[...], 0.0)
    out_ref[...] = x + res


def _pair(kvf, posg, pos16, x, wq, pw1, pb1, pw2, pb2,
          aw1, ab1, aw2, ab2, ow, ob, br=128):
    rows = x.shape[0]
    wspec = pl.BlockSpec((DIM, DIM), lambda i: (0, 0))
    bspec = pl.BlockSpec((1, DIM), lambda i: (0, 0))
    return pl.pallas_call(
        functools.partial(_pair_body, br=br),
        grid=(rows // br,),
        in_specs=[
            pl.BlockSpec((br * KNN, DIM), lambda i: (i, 0)),
            pl.BlockSpec((br * KNN, 128), lambda i: (i, 0)),
            pl.BlockSpec((br, 128), lambda i: (i, 0)),
            pl.BlockSpec((br, DIM), lambda i: (i, 0)),
            wspec,
            pl.BlockSpec((128, DIM), lambda i: (0, 0)), bspec,
            wspec, bspec, wspec, bspec, wspec, bspec, wspec, bspec,
        ],
        out_specs=pl.BlockSpec((br, DIM), lambda i: (i, 0)),
        out_shape=jax.ShapeDtypeStruct((rows, DIM), _F32),
    )(kvf, posg, pos16, x, wq, pw1, pb1, pw2, pb2,
      aw1, ab1, aw2, ab2, ow, ob)


def _fin_body(x_ref, g_ref, b_ref, fw_ref, fb_ref, out_ref, *, n):
    x = x_ref[0]
    mu = jnp.mean(x, axis=1, keepdims=True)
    var = jnp.mean((x - mu) ** 2, axis=1, keepdims=True)
    xn = (x - mu) / jnp.sqrt(var + 1e-5) * g_ref[...] + b_ref[...]
    mean = jnp.sum(xn, axis=0, keepdims=True) * _F32(1.0 / n)
    z = jnp.sum(mean * fw_ref[...], axis=1, keepdims=True) + fb_ref[...]
    out_ref[...] = jnp.broadcast_to(1.0 / (1.0 + jnp.exp(-z)), (1, 1, 128))


def _fin(x3, g2, b2, fw2, fb2):
    bsz, n, _ = x3.shape
    return pl.pallas_call(
        functools.partial(_fin_body, n=n),
        grid=(bsz,),
        in_specs=[
            pl.BlockSpec((1, n, DIM), lambda b: (b, 0, 0)),
            pl.BlockSpec((1, DIM), lambda b: (0, 0)),
            pl.BlockSpec((1, DIM), lambda b: (0, 0)),
            pl.BlockSpec((1, DIM), lambda b: (0, 0)),
            pl.BlockSpec((1, 1), lambda b: (0, 0)),
        ],
        out_specs=pl.BlockSpec((1, 1, 128), lambda b: (b, 0, 0)),
        out_shape=jax.ShapeDtypeStruct((bsz, 1, 128), _F32),
    )(x3, g2, b2, fw2, fb2)


def kernel(vector_field, pathline_src, params):
    del vector_field
    bsz, ll, kk, c = pathline_src.shape
    n = ll * kk
    p = params
    pts = pathline_src.reshape(bsz, n, c)
    pos = pts[..., :3]
    pos8 = jnp.pad(pos, ((0, 0), (0, 0), (0, 5)))
    posT = pos8.transpose(0, 2, 1)
    pos128 = jnp.pad(pos, ((0, 0), (0, 0), (0, 125))).reshape(bsz * n, 128)
    pts8 = jnp.pad(pts, ((0, 0), (0, 0), (0, 8 - c))).reshape(bsz * n, 8)

    knn = _topk(pos8, posT, n)
    idx_flat = knn.reshape(bsz * n * KNN)

    posg = _sc_gather(pos128, idx_flat)

    w8 = jnp.pad(p['emb_w'], ((0, 8 - c), (0, 0)))
    x = _emb(pts8, w8, p['emb_b'].reshape(1, DIM))

    for i in range(NLAYERS):
        kv = _kv(x, p['wk'][i], p['wv'][i])
        kvf = _sc_gather(kv, idx_flat)
        pw1 = jnp.pad(p['pos_w1'][i], ((0, 125), (0, 0)))
        x = _pair(kvf, posg, pos128, x,
                  p['wq'][i], pw1, p['pos_b1'][i].reshape(1, DIM),
                  p['pos_w2'][i], p['pos_b2'][i].reshape(1, DIM),
                  p['attn_w1'][i], p['attn_b1'][i].reshape(1, DIM),
                  p['attn_w2'][i], p['attn_b2'][i].reshape(1, DIM),
                  p['out_w'][i], p['out_b'][i].reshape(1, DIM))

    out = _fin(x.reshape(bsz, n, DIM), p['ln_g'].reshape(1, DIM),
               p['ln_b'].reshape(1, DIM), p['fc_w'].reshape(1, DIM),
               p['fc_b'].reshape(1, 1))
    return out[:, 0, :1]


# speedup vs baseline: 3.6615x; 1.0782x over previous
import functools

import jax
import jax.numpy as jnp
from jax import lax
from jax.experimental import pallas as pl
from jax.experimental.pallas import tpu as pltpu
from jax.experimental.pallas import tpu_sc as plsc

DIM = 256
KNN = 16
NLAYERS = 3

_F32 = jnp.float32


def _topk_body(pos_ref, posT_ref, out_ref, *, n):
    b = pl.program_id(0)
    prow = pos_ref[0]
    pcol = posT_ref[0]
    inner = (prow[:, 0:1] * pcol[0:1, :]
             + prow[:, 1:2] * pcol[1:2, :]
             + prow[:, 2:3] * pcol[2:3, :])
    xxr = prow[:, 0:1] ** 2 + prow[:, 1:2] ** 2 + prow[:, 2:3] ** 2
    xxc = pcol[0:1, :] ** 2 + pcol[1:2, :] ** 2 + pcol[2:3, :] ** 2
    pd = 2.0 * inner - xxr - xxc
    br = prow.shape[0]
    cols = lax.broadcasted_iota(jnp.int32, (br, n), 1)
    bits = lax.bitcast_convert_type(pd, jnp.int32)
    minint = jnp.int32(-(2 ** 31))
    skey = jnp.where(bits < 0, jnp.invert(bits) ^ minint, bits)
    ck = (skey & jnp.int32(-65536)) | (jnp.int32(n - 1) - cols)
    outs = []
    for _ in range(KNN):
        m = jnp.max(ck, axis=1, keepdims=True)
        idx = jnp.int32(n - 1) - (m & jnp.int32(0xFFFF))
        outs.append(idx)
        ck = jnp.where(ck == m, minint, ck)
    out_ref[0] = jnp.concatenate(outs, axis=1) + b * n


def _topk(pos8, posT, n, br=256):
    bsz = pos8.shape[0]
    return pl.pallas_call(
        functools.partial(_topk_body, n=n),
        grid=(bsz, n // br),
        in_specs=[
            pl.BlockSpec((1, br, 8), lambda b, i: (b, i, 0)),
            pl.BlockSpec((1, 8, n), lambda b, i: (b, 0, 0)),
        ],
        out_specs=pl.BlockSpec((1, br, KNN), lambda b, i: (b, i, 0)),
        out_shape=jax.ShapeDtypeStruct((bsz, n, KNN), jnp.int32),
    )(pos8, posT)


def _emb_body(pts_ref, w_ref, b_ref, out_ref):
    out_ref[...] = (jnp.dot(pts_ref[...], w_ref[...],
                            preferred_element_type=_F32) + b_ref[...])


def _emb(pts8, w8, b2, blk=512):
    rows = pts8.shape[0]
    return pl.pallas_call(
        _emb_body,
        grid=(rows // blk,),
        in_specs=[
            pl.BlockSpec((blk, 8), lambda i: (i, 0)),
            pl.BlockSpec((8, DIM), lambda i: (0, 0)),
            pl.BlockSpec((1, DIM), lambda i: (0, 0)),
        ],
        out_specs=pl.BlockSpec((blk, DIM), lambda i: (i, 0)),
        out_shape=jax.ShapeDtypeStruct((rows, DIM), _F32),
    )(pts8, w8, b2)


def _rtne16(f):
    u = lax.bitcast_convert_type(f, jnp.uint32)
    return (u + jnp.uint32(0x7FFF) + ((u >> 16) & jnp.uint32(1))) >> 16


def _kv_body(x_ref, wk_ref, wv_ref, out_ref):
    x = x_ref[...]
    k = jnp.dot(x, wk_ref[...], preferred_element_type=_F32)
    v = jnp.dot(x, wv_ref[...], preferred_element_type=_F32)
    packed = (_rtne16(v) << 16) | _rtne16(k)
    out_ref[...] = lax.bitcast_convert_type(packed, jnp.int32)


def _kv(x, wk, wv, blk=512):
    rows = x.shape[0]
    return pl.pallas_call(
        _kv_body,
        grid=(rows // blk,),
        in_specs=[
            pl.BlockSpec((blk, DIM), lambda i: (i, 0)),
            pl.BlockSpec((DIM, DIM), lambda i: (0, 0)),
            pl.BlockSpec((DIM, DIM), lambda i: (0, 0)),
        ],
        out_specs=pl.BlockSpec((blk, DIM), lambda i: (i, 0)),
        out_shape=jax.ShapeDtypeStruct((rows, DIM), jnp.int32),
    )(x, wk, wv)


def _sc_gather(table, idx):
    tot = idx.shape[0]
    row_shape = table.shape[1:]
    info = plsc.get_sparse_core_info()
    nw = info.num_cores * info.num_subcores
    rows_w = tot // nw
    ch = 128
    nc = rows_w // ch
    mesh = plsc.VectorSubcoreMesh(core_axis_name="c", subcore_axis_name="s")

    @functools.partial(
        pl.kernel, mesh=mesh,
        out_type=jax.ShapeDtypeStruct((tot,) + row_shape, table.dtype),
        scratch_types=[
            pltpu.VMEM((rows_w,), jnp.int32),
            pltpu.VMEM((ch,) + row_shape, table.dtype),
            pltpu.SemaphoreType.DMA,
        ],
    )
    def gk(idx_hbm, table_hbm, out_hbm, idx_v, buf, sem):
        wid = lax.axis_index("s") * info.num_cores + lax.axis_index("c")
        base = wid * rows_w
        pltpu.sync_copy(idx_hbm.at[pl.ds(base, rows_w)], idx_v)

        def body(c, carry):
            off = c * ch
            pltpu.async_copy(table_hbm.at[idx_v.at[pl.ds(off, ch)]],
                             buf, sem).wait()
            pltpu.sync_copy(buf, out_hbm.at[pl.ds(base + off, ch)])
            return carry

        lax.fori_loop(0, nc, body, 0)

    return gk(idx, table)


def _pair_body(kvf_ref, posg_ref, pos_ref, x_ref,
               wq_ref, pw1_ref, pb1_ref, pw2_ref, pb2_ref,
               aw1_ref, ab1_ref, aw2_ref, ab2_ref, ow_ref, ob_ref,
               out_ref, *, br):
    pr = br * KNN
    kw = kvf_ref[...]
    kf = lax.bitcast_convert_type(kw << 16, _F32)
    vf = lax.bitcast_convert_type(kw & jnp.int32(-65536), _F32)
    rel3 = (posg_ref[...].reshape(br, KNN, 128)
            - pos_ref[...].reshape(br, 1, 128))
    rel = rel3.reshape(pr, 128)
    h = jnp.maximum(jnp.dot(rel, pw1_ref[...],
                            preferred_element_type=_F32) + pb1_ref[...], 0.0)
    pe = jnp.dot(h, pw2_ref[...], preferred_element_type=_F32) + pb2_ref[...]
    x = x_ref[...]
    q = jnp.dot(x, wq_ref[...], preferred_element_type=_F32)
    energy = (q.reshape(br, 1, DIM)
              - kf.reshape(br, KNN, DIM)
              + pe.reshape(br, KNN, DIM)).reshape(pr, DIM)
    a = jnp.maximum(jnp.dot(energy, aw1_ref[...],
                            preferred_element_type=_F32) + ab1_ref[...], 0.0)
    a = jnp.dot(a, aw2_ref[...], preferred_element_type=_F32) + ab2_ref[...]
    a3 = a.reshape(br, KNN, DIM)
    m = jnp.max(a3, axis=1, keepdims=True)
    e = jnp.exp(a3 - m)
    w = e / jnp.sum(e, axis=1, keepdims=True)
    out = jnp.sum(w * vf.reshape(br, KNN, DIM), axis=1)
    res = jnp.maximum(jnp.dot(out, ow_ref[...],
                              preferred_element_type=_F32) + ob_ref[...], 0.0)
    out_ref[...] = x + res


def _pair(kvf, posg, pos16, x, wq, pw1, pb1, pw2, pb2,
          aw1, ab1, aw2, ab2, ow, ob, br=128):
    rows = x.shape[0]
    wspec = pl.BlockSpec((DIM, DIM), lambda i: (0, 0))
    bspec = pl.BlockSpec((1, DIM), lambda i: (0, 0))
    return pl.pallas_call(
        functools.partial(_pair_body, br=br),
        grid=(rows // br,),
        in_specs=[
            pl.BlockSpec((br * KNN, DIM), lambda i: (i, 0)),
            pl.BlockSpec((br * KNN, 128), lambda i: (i, 0)),
            pl.BlockSpec((br, 128), lambda i: (i, 0)),
            pl.BlockSpec((br, DIM), lambda i: (i, 0)),
            wspec,
            pl.BlockSpec((128, DIM), lambda i: (0, 0)), bspec,
            wspec, bspec, wspec, bspec, wspec, bspec, wspec, bspec,
        ],
        out_specs=pl.BlockSpec((br, DIM), lambda i: (i, 0)),
        out_shape=jax.ShapeDtypeStruct((rows, DIM), _F32),
    )(kvf, posg, pos16, x, wq, pw1, pb1, pw2, pb2,
      aw1, ab1, aw2, ab2, ow, ob)


def _fin_body(x_ref, g_ref, b_ref, fw_ref, fb_ref, out_ref, *, n):
    x = x_ref[0]
    mu = jnp.mean(x, axis=1, keepdims=True)
    var = jnp.mean((x - mu) ** 2, axis=1, keepdims=True)
    xn = (x - mu) / jnp.sqrt(var + 1e-5) * g_ref[...] + b_ref[...]
    mean = jnp.sum(xn, axis=0, keepdims=True) * _F32(1.0 / n)
    z = jnp.sum(mean * fw_ref[...], axis=1, keepdims=True) + fb_ref[...]
    out_ref[...] = jnp.broadcast_to(1.0 / (1.0 + jnp.exp(-z)), (1, 1, 128))


def _fin(x3, g2, b2, fw2, fb2):
    bsz, n, _ = x3.shape
    return pl.pallas_call(
        functools.partial(_fin_body, n=n),
        grid=(bsz,),
        in_specs=[
            pl.BlockSpec((1, n, DIM), lambda b: (b, 0, 0)),
            pl.BlockSpec((1, DIM), lambda b: (0, 0)),
            pl.BlockSpec((1, DIM), lambda b: (0, 0)),
            pl.BlockSpec((1, DIM), lambda b: (0, 0)),
            pl.BlockSpec((1, 1), lambda b: (0, 0)),
        ],
        out_specs=pl.BlockSpec((1, 1, 128), lambda b: (b, 0, 0)),
        out_shape=jax.ShapeDtypeStruct((bsz, 1, 128), _F32),
    )(x3, g2, b2, fw2, fb2)


def kernel(vector_field, pathline_src, params):
    del vector_field
    bsz, ll, kk, c = pathline_src.shape
    n = ll * kk
    p = params
    pts = pathline_src.reshape(bsz, n, c)
    pos = pts[..., :3]
    pos8 = jnp.pad(pos, ((0, 0), (0, 0), (0, 5)))
    posT = pos8.transpose(0, 2, 1)
    pos128 = jnp.pad(pos, ((0, 0), (0, 0), (0, 125))).reshape(bsz * n, 128)
    pts8 = jnp.pad(pts, ((0, 0), (0, 0), (0, 8 - c))).reshape(bsz * n, 8)

    knn = _topk(pos8, posT, n)
    idx_flat = knn.reshape(bsz * n * KNN)
    nh = 2
    rows = bsz * n
    prows = rows * KNN
    idx_h = [lax.slice(idx_flat, (h * prows // nh,),
                       ((h + 1) * prows // nh,)) for h in range(nh)]

    w8 = jnp.pad(p['emb_w'], ((0, 8 - c), (0, 0)))
    x = _emb(pts8, w8, p['emb_b'].reshape(1, DIM))

    posg_h = [_sc_gather(pos128, ih) for ih in idx_h]
    pos128_h = [lax.slice(pos128, (h * rows // nh, 0),
                          ((h + 1) * rows // nh, 128)) for h in range(nh)]

    for i in range(NLAYERS):
        kv = _kv(x, p['wk'][i], p['wv'][i])
        kvf_h = [_sc_gather(kv, ih) for ih in idx_h]
        pw1 = jnp.pad(p['pos_w1'][i], ((0, 125), (0, 0)))
        wargs = (p['wq'][i], pw1, p['pos_b1'][i].reshape(1, DIM),
                 p['pos_w2'][i], p['pos_b2'][i].reshape(1, DIM),
                 p['attn_w1'][i], p['attn_b1'][i].reshape(1, DIM),
                 p['attn_w2'][i], p['attn_b2'][i].reshape(1, DIM),
                 p['out_w'][i], p['out_b'][i].reshape(1, DIM))
        x_h = [lax.slice(x, (h * rows // nh, 0),
                         ((h + 1) * rows // nh, DIM)) for h in range(nh)]
        x = jnp.concatenate(
            [_pair(kvf_h[h], posg_h[h], pos128_h[h], x_h[h], *wargs)
             for h in range(nh)], axis=0)

    out = _fin(x.reshape(bsz, n, DIM), p['ln_g'].reshape(1, DIM),
               p['ln_b'].reshape(1, DIM), p['fc_w'].reshape(1, DIM),
               p['fc_b'].reshape(1, 1))
    return out[:, 0, :1]


# speedup vs baseline: 3.7645x; 1.0281x over previous
import functools

import jax
import jax.numpy as jnp
from jax import lax
from jax.experimental import pallas as pl
from jax.experimental.pallas import tpu as pltpu
from jax.experimental.pallas import tpu_sc as plsc

DIM = 256
KNN = 16
NLAYERS = 3

_F32 = jnp.float32


def _topk_body(pos_ref, posT_ref, out_ref, *, n):
    b = pl.program_id(0)
    prow = pos_ref[0]
    pcol = posT_ref[0]
    inner = (prow[:, 0:1] * pcol[0:1, :]
             + prow[:, 1:2] * pcol[1:2, :]
             + prow[:, 2:3] * pcol[2:3, :])
    xxr = prow[:, 0:1] ** 2 + prow[:, 1:2] ** 2 + prow[:, 2:3] ** 2
    xxc = pcol[0:1, :] ** 2 + pcol[1:2, :] ** 2 + pcol[2:3, :] ** 2
    pd = 2.0 * inner - xxr - xxc
    br = prow.shape[0]
    cols = lax.broadcasted_iota(jnp.int32, (br, n), 1)
    bits = lax.bitcast_convert_type(pd, jnp.int32)
    minint = jnp.int32(-(2 ** 31))
    skey = jnp.where(bits < 0, jnp.invert(bits) ^ minint, bits)
    ck = (skey & jnp.int32(-65536)) | (jnp.int32(n - 1) - cols)
    outs = []
    for _ in range(KNN):
        m = jnp.max(ck, axis=1, keepdims=True)
        idx = jnp.int32(n - 1) - (m & jnp.int32(0xFFFF))
        outs.append(idx)
        ck = jnp.where(ck == m, minint, ck)
    out_ref[0] = jnp.concatenate(outs, axis=1) + b * n


def _topk(pos8, posT, n, br=256):
    bsz = pos8.shape[0]
    return pl.pallas_call(
        functools.partial(_topk_body, n=n),
        grid=(bsz, n // br),
        in_specs=[
            pl.BlockSpec((1, br, 8), lambda b, i: (b, i, 0)),
            pl.BlockSpec((1, 8, n), lambda b, i: (b, 0, 0)),
        ],
        out_specs=pl.BlockSpec((1, br, KNN), lambda b, i: (b, i, 0)),
        out_shape=jax.ShapeDtypeStruct((bsz, n, KNN), jnp.int32),
    )(pos8, posT)


def _emb_body(pts_ref, w_ref, b_ref, out_ref):
    out_ref[...] = (jnp.dot(pts_ref[...], w_ref[...],
                            preferred_element_type=_F32) + b_ref[...])


def _emb(pts8, w8, b2, blk=512):
    rows = pts8.shape[0]
    return pl.pallas_call(
        _emb_body,
        grid=(rows // blk,),
        in_specs=[
            pl.BlockSpec((blk, 8), lambda i: (i, 0)),
            pl.BlockSpec((8, DIM), lambda i: (0, 0)),
            pl.BlockSpec((1, DIM), lambda i: (0, 0)),
        ],
        out_specs=pl.BlockSpec((blk, DIM), lambda i: (i, 0)),
        out_shape=jax.ShapeDtypeStruct((rows, DIM), _F32),
    )(pts8, w8, b2)


def _rtne16(f):
    u = lax.bitcast_convert_type(f, jnp.uint32)
    return (u + jnp.uint32(0x7FFF) + ((u >> 16) & jnp.uint32(1))) >> 16


def _kv_body(x_ref, wk_ref, wv_ref, out_ref):
    x = x_ref[...]
    k = jnp.dot(x, wk_ref[...], preferred_element_type=_F32)
    v = jnp.dot(x, wv_ref[...], preferred_element_type=_F32)
    packed = (_rtne16(v) << 16) | _rtne16(k)
    out_ref[...] = lax.bitcast_convert_type(packed, jnp.int32)


def _kv(x, wk, wv, blk=512):
    rows = x.shape[0]
    return pl.pallas_call(
        _kv_body,
        grid=(rows // blk,),
        in_specs=[
            pl.BlockSpec((blk, DIM), lambda i: (i, 0)),
            pl.BlockSpec((DIM, DIM), lambda i: (0, 0)),
            pl.BlockSpec((DIM, DIM), lambda i: (0, 0)),
        ],
        out_specs=pl.BlockSpec((blk, DIM), lambda i: (i, 0)),
        out_shape=jax.ShapeDtypeStruct((rows, DIM), jnp.int32),
    )(x, wk, wv)


def _sc_gather(table, idx):
    tot = idx.shape[0]
    row_shape = table.shape[1:]
    info = plsc.get_sparse_core_info()
    nw = info.num_cores * info.num_subcores
    rows_w = tot // nw
    ch = 128
    nc = rows_w // ch
    mesh = plsc.VectorSubcoreMesh(core_axis_name="c", subcore_axis_name="s")

    @functools.partial(
        pl.kernel, mesh=mesh,
        out_type=jax.ShapeDtypeStruct((tot,) + row_shape, table.dtype),
        scratch_types=[
            pltpu.VMEM((rows_w,), jnp.int32),
            pltpu.VMEM((ch,) + row_shape, table.dtype),
            pltpu.SemaphoreType.DMA,
        ],
    )
    def gk(idx_hbm, table_hbm, out_hbm, idx_v, buf, sem):
        wid = lax.axis_index("s") * info.num_cores + lax.axis_index("c")
        base = wid * rows_w
        pltpu.sync_copy(idx_hbm.at[pl.ds(base, rows_w)], idx_v)

        def body(c, carry):
            off = c * ch
            pltpu.async_copy(table_hbm.at[idx_v.at[pl.ds(off, ch)]],
                             buf, sem).wait()
            pltpu.sync_copy(buf, out_hbm.at[pl.ds(base + off, ch)])
            return carry

        lax.fori_loop(0, nc, body, 0)

    return gk(idx, table)


def _pair_body(kvf_ref, posg_ref, pos_ref, x_ref,
               wq_ref, pw1_ref, pb1_ref, pw2_ref, pb2_ref,
               aw1_ref, ab1_ref, aw2_ref, ab2_ref, ow_ref, ob_ref,
               out_ref, *, br):
    pr = br * KNN
    kw = kvf_ref[...]
    kf = lax.bitcast_convert_type(kw << 16, _F32)
    vf = lax.bitcast_convert_type(kw & jnp.int32(-65536), _F32)
    rel3 = (posg_ref[...].reshape(br, KNN, 128)
            - pos_ref[...].reshape(br, 1, 128))
    rel = rel3.reshape(pr, 128)
    h = jnp.maximum(jnp.dot(rel, pw1_ref[...],
                            preferred_element_type=_F32) + pb1_ref[...], 0.0)
    pe = jnp.dot(h, pw2_ref[...], preferred_element_type=_F32) + pb2_ref[...]
    x = x_ref[...]
    q = jnp.dot(x, wq_ref[...], preferred_element_type=_F32)
    energy = (q.reshape(br, 1, DIM)
              - kf.reshape(br, KNN, DIM)
              + pe.reshape(br, KNN, DIM)).reshape(pr, DIM)
    a = jnp.maximum(jnp.dot(energy, aw1_ref[...],
                            preferred_element_type=_F32) + ab1_ref[...], 0.0)
    a = jnp.dot(a, aw2_ref[...], preferred_element_type=_F32) + ab2_ref[...]
    a3 = a.reshape(br, KNN, DIM)
    m = jnp.max(a3, axis=1, keepdims=True)
    e = jnp.exp(a3 - m)
    w = e / jnp.sum(e, axis=1, keepdims=True)
    out = jnp.sum(w * vf.reshape(br, KNN, DIM), axis=1)
    res = jnp.maximum(jnp.dot(out, ow_ref[...],
                              preferred_element_type=_F32) + ob_ref[...], 0.0)
    out_ref[...] = x + res


def _pair(kvf, posg, pos16, x, wq, pw1, pb1, pw2, pb2,
          aw1, ab1, aw2, ab2, ow, ob, br=128):
    rows = x.shape[0]
    wspec = pl.BlockSpec((DIM, DIM), lambda i: (0, 0))
    bspec = pl.BlockSpec((1, DIM), lambda i: (0, 0))
    return pl.pallas_call(
        functools.partial(_pair_body, br=br),
        grid=(rows // br,),
        in_specs=[
            pl.BlockSpec((br * KNN, DIM), lambda i: (i, 0)),
            pl.BlockSpec((br * KNN, 128), lambda i: (i, 0)),
            pl.BlockSpec((br, 128), lambda i: (i, 0)),
            pl.BlockSpec((br, DIM), lambda i: (i, 0)),
            wspec,
            pl.BlockSpec((128, DIM), lambda i: (0, 0)), bspec,
            wspec, bspec, wspec, bspec, wspec, bspec, wspec, bspec,
        ],
        out_specs=pl.BlockSpec((br, DIM), lambda i: (i, 0)),
        out_shape=jax.ShapeDtypeStruct((rows, DIM), _F32),
    )(kvf, posg, pos16, x, wq, pw1, pb1, pw2, pb2,
      aw1, ab1, aw2, ab2, ow, ob)


def _fin_body(x_ref, g_ref, b_ref, fw_ref, fb_ref, out_ref, *, n):
    x = x_ref[0]
    mu = jnp.mean(x, axis=1, keepdims=True)
    var = jnp.mean((x - mu) ** 2, axis=1, keepdims=True)
    xn = (x - mu) / jnp.sqrt(var + 1e-5) * g_ref[...] + b_ref[...]
    mean = jnp.sum(xn, axis=0, keepdims=True) * _F32(1.0 / n)
    z = jnp.sum(mean * fw_ref[...], axis=1, keepdims=True) + fb_ref[...]
    out_ref[...] = jnp.broadcast_to(1.0 / (1.0 + jnp.exp(-z)), (1, 1, 128))


def _fin(x3, g2, b2, fw2, fb2):
    bsz, n, _ = x3.shape
    return pl.pallas_call(
        functools.partial(_fin_body, n=n),
        grid=(bsz,),
        in_specs=[
            pl.BlockSpec((1, n, DIM), lambda b: (b, 0, 0)),
            pl.BlockSpec((1, DIM), lambda b: (0, 0)),
            pl.BlockSpec((1, DIM), lambda b: (0, 0)),
            pl.BlockSpec((1, DIM), lambda b: (0, 0)),
            pl.BlockSpec((1, 1), lambda b: (0, 0)),
        ],
        out_specs=pl.BlockSpec((1, 1, 128), lambda b: (b, 0, 0)),
        out_shape=jax.ShapeDtypeStruct((bsz, 1, 128), _F32),
    )(x3, g2, b2, fw2, fb2)


def kernel(vector_field, pathline_src, params):
    del vector_field
    bsz, ll, kk, c = pathline_src.shape
    n = ll * kk
    p = params
    pts = pathline_src.reshape(bsz, n, c)
    pos = pts[..., :3]
    pos8 = jnp.pad(pos, ((0, 0), (0, 0), (0, 5)))
    posT = pos8.transpose(0, 2, 1)
    pos128 = jnp.pad(pos, ((0, 0), (0, 0), (0, 125))).reshape(bsz * n, 128)
    pts8 = jnp.pad(pts, ((0, 0), (0, 0), (0, 8 - c))).reshape(bsz * n, 8)

    knn = _topk(pos8, posT, n)
    idx_flat = knn.reshape(bsz * n * KNN)
    nh = 4
    rows = bsz * n
    prows = rows * KNN
    idx_h = [lax.slice(idx_flat, (h * prows // nh,),
                       ((h + 1) * prows // nh,)) for h in range(nh)]

    w8 = jnp.pad(p['emb_w'], ((0, 8 - c), (0, 0)))
    x = _emb(pts8, w8, p['emb_b'].reshape(1, DIM))

    posg_h = [_sc_gather(pos128, ih) for ih in idx_h]
    pos128_h = [lax.slice(pos128, (h * rows // nh, 0),
                          ((h + 1) * rows // nh, 128)) for h in range(nh)]

    for i in range(NLAYERS):
        kv = _kv(x, p['wk'][i], p['wv'][i])
        kvf_h = [_sc_gather(kv, ih) for ih in idx_h]
        pw1 = jnp.pad(p['pos_w1'][i], ((0, 125), (0, 0)))
        wargs = (p['wq'][i], pw1, p['pos_b1'][i].reshape(1, DIM),
                 p['pos_w2'][i], p['pos_b2'][i].reshape(1, DIM),
                 p['attn_w1'][i], p['attn_b1'][i].reshape(1, DIM),
                 p['attn_w2'][i], p['attn_b2'][i].reshape(1, DIM),
                 p['out_w'][i], p['out_b'][i].reshape(1, DIM))
        x_h = [lax.slice(x, (h * rows // nh, 0),
                         ((h + 1) * rows // nh, DIM)) for h in range(nh)]
        x = jnp.concatenate(
            [_pair(kvf_h[h], posg_h[h], pos128_h[h], x_h[h], *wargs)
             for h in range(nh)], axis=0)

    out = _fin(x.reshape(bsz, n, DIM), p['ln_g'].reshape(1, DIM),
               p['ln_b'].reshape(1, DIM), p['fc_w'].reshape(1, DIM),
               p['fc_b'].reshape(1, 1))
    return out[:, 0, :1]


# speedup vs baseline: 3.7742x; 1.0026x over previous
import functools

import jax
import jax.numpy as jnp
from jax import lax
from jax.experimental import pallas as pl
from jax.experimental.pallas import tpu as pltpu
from jax.experimental.pallas import tpu_sc as plsc

DIM = 256
KNN = 16
NLAYERS = 3

_F32 = jnp.float32


def _topk_body(pos_ref, posT_ref, out_ref, *, n):
    b = pl.program_id(0)
    prow = pos_ref[0]
    pcol = posT_ref[0]
    inner = (prow[:, 0:1] * pcol[0:1, :]
             + prow[:, 1:2] * pcol[1:2, :]
             + prow[:, 2:3] * pcol[2:3, :])
    xxr = prow[:, 0:1] ** 2 + prow[:, 1:2] ** 2 + prow[:, 2:3] ** 2
    xxc = pcol[0:1, :] ** 2 + pcol[1:2, :] ** 2 + pcol[2:3, :] ** 2
    pd = 2.0 * inner - xxr - xxc
    br = prow.shape[0]
    cols = lax.broadcasted_iota(jnp.int32, (br, n), 1)
    bits = lax.bitcast_convert_type(pd, jnp.int32)
    minint = jnp.int32(-(2 ** 31))
    skey = jnp.where(bits < 0, jnp.invert(bits) ^ minint, bits)
    ck = (skey & jnp.int32(-65536)) | (jnp.int32(n - 1) - cols)
    outs = []
    for _ in range(KNN):
        m = jnp.max(ck, axis=1, keepdims=True)
        idx = jnp.int32(n - 1) - (m & jnp.int32(0xFFFF))
        outs.append(idx)
        ck = jnp.where(ck == m, minint, ck)
    out_ref[0] = jnp.concatenate(outs, axis=1) + b * n


def _topk(pos8, posT, n, br=256):
    bsz = pos8.shape[0]
    return pl.pallas_call(
        functools.partial(_topk_body, n=n),
        grid=(bsz, n // br),
        in_specs=[
            pl.BlockSpec((1, br, 8), lambda b, i: (b, i, 0)),
            pl.BlockSpec((1, 8, n), lambda b, i: (b, 0, 0)),
        ],
        out_specs=pl.BlockSpec((1, br, KNN), lambda b, i: (b, i, 0)),
        out_shape=jax.ShapeDtypeStruct((bsz, n, KNN), jnp.int32),
    )(pos8, posT)


def _emb_body(pts_ref, w_ref, b_ref, out_ref):
    out_ref[...] = (jnp.dot(pts_ref[...], w_ref[...],
                            preferred_element_type=_F32) + b_ref[...])


def _emb(pts8, w8, b2, blk=512):
    rows = pts8.shape[0]
    return pl.pallas_call(
        _emb_body,
        grid=(rows // blk,),
        in_specs=[
            pl.BlockSpec((blk, 8), lambda i: (i, 0)),
            pl.BlockSpec((8, DIM), lambda i: (0, 0)),
            pl.BlockSpec((1, DIM), lambda i: (0, 0)),
        ],
        out_specs=pl.BlockSpec((blk, DIM), lambda i: (i, 0)),
        out_shape=jax.ShapeDtypeStruct((rows, DIM), _F32),
    )(pts8, w8, b2)


def _rtne16(f):
    u = lax.bitcast_convert_type(f, jnp.uint32)
    return (u + jnp.uint32(0x7FFF) + ((u >> 16) & jnp.uint32(1))) >> 16


def _kv_body(x_ref, wk_ref, wv_ref, out_ref):
    x = x_ref[...]
    k = jnp.dot(x, wk_ref[...], preferred_element_type=_F32)
    v = jnp.dot(x, wv_ref[...], preferred_element_type=_F32)
    packed = (_rtne16(v) << 16) | _rtne16(k)
    out_ref[...] = lax.bitcast_convert_type(packed, jnp.int32)


def _kv(x, wk, wv, blk=512):
    rows = x.shape[0]
    return pl.pallas_call(
        _kv_body,
        grid=(rows // blk,),
        in_specs=[
            pl.BlockSpec((blk, DIM), lambda i: (i, 0)),
            pl.BlockSpec((DIM, DIM), lambda i: (0, 0)),
            pl.BlockSpec((DIM, DIM), lambda i: (0, 0)),
        ],
        out_specs=pl.BlockSpec((blk, DIM), lambda i: (i, 0)),
        out_shape=jax.ShapeDtypeStruct((rows, DIM), jnp.int32),
    )(x, wk, wv)


def _sc_gather(table, idx):
    tot = idx.shape[0]
    row_shape = table.shape[1:]
    info = plsc.get_sparse_core_info()
    nw = info.num_cores * info.num_subcores
    rows_w = tot // nw
    ch = 128
    nc = rows_w // ch
    mesh = plsc.VectorSubcoreMesh(core_axis_name="c", subcore_axis_name="s")

    @functools.partial(
        pl.kernel, mesh=mesh,
        out_type=jax.ShapeDtypeStruct((tot,) + row_shape, table.dtype),
        scratch_types=[
            pltpu.VMEM((rows_w,), jnp.int32),
            pltpu.VMEM((ch,) + row_shape, table.dtype),
            pltpu.SemaphoreType.DMA,
        ],
    )
    def gk(idx_hbm, table_hbm, out_hbm, idx_v, buf, sem):
        wid = lax.axis_index("s") * info.num_cores + lax.axis_index("c")
        base = wid * rows_w
        pltpu.sync_copy(idx_hbm.at[pl.ds(base, rows_w)], idx_v)

        def body(c, carry):
            off = c * ch
            pltpu.async_copy(table_hbm.at[idx_v.at[pl.ds(off, ch)]],
                             buf, sem).wait()
            pltpu.sync_copy(buf, out_hbm.at[pl.ds(base + off, ch)])
            return carry

        lax.fori_loop(0, nc, body, 0)

    return gk(idx, table)


def _pair_body(kvf_ref, posg_ref, pos_ref, x_ref,
               wq_ref, pw1_ref, pb1_ref, pw2_ref, pb2_ref,
               aw1_ref, ab1_ref, aw2_ref, ab2_ref, ow_ref, ob_ref,
               out_ref, *, br):
    pr = br * KNN
    kw = kvf_ref[...]
    kf = lax.bitcast_convert_type(kw << 16, _F32)
    vf = lax.bitcast_convert_type(kw & jnp.int32(-65536), _F32)
    rel3 = (posg_ref[...].reshape(br, KNN, 128)
            - pos_ref[...].reshape(br, 1, 128))
    rel = rel3.reshape(pr, 128)
    h = jnp.maximum(jnp.dot(rel, pw1_ref[...],
                            preferred_element_type=_F32) + pb1_ref[...], 0.0)
    pe = jnp.dot(h, pw2_ref[...], preferred_element_type=_F32) + pb2_ref[...]
    x = x_ref[...]
    q = jnp.dot(x, wq_ref[...], preferred_element_type=_F32)
    energy = (q.reshape(br, 1, DIM)
              - kf.reshape(br, KNN, DIM)
              + pe.reshape(br, KNN, DIM)).reshape(pr, DIM)
    a = jnp.maximum(jnp.dot(energy, aw1_ref[...],
                            preferred_element_type=_F32) + ab1_ref[...], 0.0)
    a = jnp.dot(a, aw2_ref[...], preferred_element_type=_F32) + ab2_ref[...]
    a3 = a.reshape(br, KNN, DIM)
    m = jnp.max(a3, axis=1, keepdims=True)
    e = jnp.exp(a3 - m)
    s = jnp.sum(e, axis=1)
    out = jnp.sum(e * vf.reshape(br, KNN, DIM), axis=1) / s
    res = jnp.maximum(jnp.dot(out, ow_ref[...],
                              preferred_element_type=_F32) + ob_ref[...], 0.0)
    out_ref[...] = x + res


def _pair(kvf, posg, pos16, x, wq, pw1, pb1, pw2, pb2,
          aw1, ab1, aw2, ab2, ow, ob, br=128):
    rows = x.shape[0]
    wspec = pl.BlockSpec((DIM, DIM), lambda i: (0, 0))
    bspec = pl.BlockSpec((1, DIM), lambda i: (0, 0))
    return pl.pallas_call(
        functools.partial(_pair_body, br=br),
        grid=(rows // br,),
        in_specs=[
            pl.BlockSpec((br * KNN, DIM), lambda i: (i, 0)),
            pl.BlockSpec((br * KNN, 128), lambda i: (i, 0)),
            pl.BlockSpec((br, 128), lambda i: (i, 0)),
            pl.BlockSpec((br, DIM), lambda i: (i, 0)),
            wspec,
            pl.BlockSpec((128, DIM), lambda i: (0, 0)), bspec,
            wspec, bspec, wspec, bspec, wspec, bspec, wspec, bspec,
        ],
        out_specs=pl.BlockSpec((br, DIM), lambda i: (i, 0)),
        out_shape=jax.ShapeDtypeStruct((rows, DIM), _F32),
    )(kvf, posg, pos16, x, wq, pw1, pb1, pw2, pb2,
      aw1, ab1, aw2, ab2, ow, ob)


def _fin_body(x_ref, g_ref, b_ref, fw_ref, fb_ref, out_ref, *, n):
    x = x_ref[0]
    mu = jnp.mean(x, axis=1, keepdims=True)
    var = jnp.mean((x - mu) ** 2, axis=1, keepdims=True)
    xn = (x - mu) / jnp.sqrt(var + 1e-5) * g_ref[...] + b_ref[...]
    mean = jnp.sum(xn, axis=0, keepdims=True) * _F32(1.0 / n)
    z = jnp.sum(mean * fw_ref[...], axis=1, keepdims=True) + fb_ref[...]
    out_ref[...] = jnp.broadcast_to(1.0 / (1.0 + jnp.exp(-z)), (1, 1, 128))


def _fin(x3, g2, b2, fw2, fb2):
    bsz, n, _ = x3.shape
    return pl.pallas_call(
        functools.partial(_fin_body, n=n),
        grid=(bsz,),
        in_specs=[
            pl.BlockSpec((1, n, DIM), lambda b: (b, 0, 0)),
            pl.BlockSpec((1, DIM), lambda b: (0, 0)),
            pl.BlockSpec((1, DIM), lambda b: (0, 0)),
            pl.BlockSpec((1, DIM), lambda b: (0, 0)),
            pl.BlockSpec((1, 1), lambda b: (0, 0)),
        ],
        out_specs=pl.BlockSpec((1, 1, 128), lambda b: (b, 0, 0)),
        out_shape=jax.ShapeDtypeStruct((bsz, 1, 128), _F32),
    )(x3, g2, b2, fw2, fb2)


def kernel(vector_field, pathline_src, params):
    del vector_field
    bsz, ll, kk, c = pathline_src.shape
    n = ll * kk
    p = params
    pts = pathline_src.reshape(bsz, n, c)
    pos = pts[..., :3]
    pos8 = jnp.pad(pos, ((0, 0), (0, 0), (0, 5)))
    posT = pos8.transpose(0, 2, 1)
    pos128 = jnp.pad(pos, ((0, 0), (0, 0), (0, 125))).reshape(bsz * n, 128)
    pts8 = jnp.pad(pts, ((0, 0), (0, 0), (0, 8 - c))).reshape(bsz * n, 8)

    knn = _topk(pos8, posT, n)
    idx_flat = knn.reshape(bsz * n * KNN)
    nh = 4
    rows = bsz * n
    prows = rows * KNN
    idx_h = [lax.slice(idx_flat, (h * prows // nh,),
                       ((h + 1) * prows // nh,)) for h in range(nh)]

    w8 = jnp.pad(p['emb_w'], ((0, 8 - c), (0, 0)))
    x = _emb(pts8, w8, p['emb_b'].reshape(1, DIM))

    posg_h = [_sc_gather(pos128, ih) for ih in idx_h]
    pos128_h = [lax.slice(pos128, (h * rows // nh, 0),
                          ((h + 1) * rows // nh, 128)) for h in range(nh)]

    for i in range(NLAYERS):
        kv = _kv(x, p['wk'][i], p['wv'][i])
        kvf_h = [_sc_gather(kv, ih) for ih in idx_h]
        pw1 = jnp.pad(p['pos_w1'][i], ((0, 125), (0, 0)))
        wargs = (p['wq'][i], pw1, p['pos_b1'][i].reshape(1, DIM),
                 p['pos_w2'][i], p['pos_b2'][i].reshape(1, DIM),
                 p['attn_w1'][i], p['attn_b1'][i].reshape(1, DIM),
                 p['attn_w2'][i], p['attn_b2'][i].reshape(1, DIM),
                 p['out_w'][i], p['out_b'][i].reshape(1, DIM))
        x_h = [lax.slice(x, (h * rows // nh, 0),
                         ((h + 1) * rows // nh, DIM)) for h in range(nh)]
        x = jnp.concatenate(
            [_pair(kvf_h[h], posg_h[h], pos128_h[h], x_h[h], *wargs)
             for h in range(nh)], axis=0)

    out = _fin(x.reshape(bsz, n, DIM), p['ln_g'].reshape(1, DIM),
               p['ln_b'].reshape(1, DIM), p['fc_w'].reshape(1, DIM),
               p['fc_b'].reshape(1, 1))
    return out[:, 0, :1]
